# Initial kernel scaffold; baseline (speedup 1.0000x reference)
import jax
import jax.numpy as jnp
from jax.experimental import pallas as pl


def kernel(ent_emds, rel_emds, time_table, edge_index, edge_type, edge_time, w_in1, w_out1, w_loop1, w_rel1, loop_rel1, bias1, w_in2, w_out2, w_loop2, w_rel2, loop_rel2, bias2):
    raise NotImplementedError("write your pallas kernel here")


# speedup vs baseline: 1.9324x; 1.9324x over previous
import jax
import jax.numpy as jnp
from jax import lax
from jax.experimental import pallas as pl

N_ENT = 10000
N_REL = 240
D = 128

_RB = 2000


def _layer_body(ain_ref, aout_ref, x_ref, norm_ref, win_ref, wout_ref,
                wloop_ref, lrel_ref, bias_ref, xn_ref, xs_ref):
    agg = (jnp.dot(ain_ref[...], win_ref[...], preferred_element_type=jnp.float32)
           + jnp.dot(aout_ref[...], wout_ref[...], preferred_element_type=jnp.float32))
    loop_msg = jnp.dot(x_ref[...] * lrel_ref[...], wloop_ref[...],
                       preferred_element_type=jnp.float32)
    xn = (norm_ref[...] * agg + loop_msg) * (1.0 / 3.0) + bias_ref[...]
    xn_ref[...] = xn
    xs_ref[...] = xn * norm_ref[...]


def _layer_call(ain, aout, x, normc, w_in, w_out, w_loop, loop_rel, bias):
    grid = (N_ENT // _RB,)
    bs = pl.BlockSpec((_RB, D), lambda i: (i, 0))
    bw = pl.BlockSpec((D, D), lambda i: (0, 0))
    b1 = pl.BlockSpec((1, D), lambda i: (0, 0))
    return pl.pallas_call(
        _layer_body,
        grid=grid,
        in_specs=[bs, bs, bs, bs, bw, bw, bw, b1, b1],
        out_specs=[bs, bs],
        out_shape=[jax.ShapeDtypeStruct((N_ENT, D), jnp.float32),
                   jax.ShapeDtypeStruct((N_ENT, D), jnp.float32)],
    )(ain, aout, x, normc, w_in, w_out, w_loop, loop_rel,
      jnp.reshape(bias, (1, D)))


def _rels_body(r_ref, w1_ref, w2_ref, r1_ref, r2_ref):
    r1 = jnp.dot(r_ref[...], w1_ref[...], preferred_element_type=jnp.float32)
    r1_ref[...] = r1
    r2_ref[...] = jnp.dot(r1, w2_ref[...], preferred_element_type=jnp.float32)


def _rels_call(r, w1, w2):
    nr = 2 * N_REL
    return pl.pallas_call(
        _rels_body,
        out_shape=[jax.ShapeDtypeStruct((nr, D), jnp.float32),
                   jax.ShapeDtypeStruct((nr, D), jnp.float32)],
    )(r, w1, w2)


def kernel(ent_emds, rel_emds, time_table, edge_index, edge_type, edge_time,
           w_in1, w_out1, w_loop1, w_rel1, loop_rel1, bias1,
           w_in2, w_out2, w_loop2, w_rel2, loop_rel2, bias2):
    src = edge_index[0]
    dst = edge_index[1]

    deg = jnp.zeros((N_ENT,), jnp.float32).at[dst].add(1.0)
    normj = jnp.where(deg > 0, lax.rsqrt(jnp.maximum(deg, 1.0)), 0.0)
    normc = jnp.broadcast_to(normj[:, None], (N_ENT, D))

    sb = lax.shift_right_logical(edge_type - N_REL, 31)
    sidx = (1 - sb) * N_ENT + dst
    time_emd = time_table[edge_time]

    def edge_pass(xs, relt):
        m = xs[src] * (relt[edge_type] + time_emd)
        acc = jnp.zeros((2 * N_ENT, D), jnp.float32).at[sidx].add(m)
        return acc[:N_ENT], acc[N_ENT:]

    r1, r2 = _rels_call(rel_emds, w_rel1, w_rel2)

    relt1 = jnp.concatenate([rel_emds, loop_rel1], axis=0)
    a1, b1 = edge_pass(ent_emds * normc, relt1)
    x1, xs1 = _layer_call(a1, b1, ent_emds, normc,
                          w_in1, w_out1, w_loop1, loop_rel1, bias1)

    relt2 = jnp.concatenate([r1, loop_rel2], axis=0)
    a2, b2 = edge_pass(xs1, relt2)
    x2, _ = _layer_call(a2, b2, x1, normc,
                        w_in2, w_out2, w_loop2, loop_rel2, bias2)

    return (x2, r2)
